# baseline (speedup 1.0000x reference)
import functools

import jax
import jax.numpy as jnp
from jax import lax
from jax.experimental import pallas as pl
from jax.experimental.pallas import tpu as pltpu
from jax.experimental.pallas import tpu_sc as plsc


def kernel(x, emb_table, pos_embd):
    B, S = x.shape
    V, D = emb_table.shape
    info = plsc.get_sparse_core_info()
    NC, NS, L = info.num_cores, info.num_subcores, info.num_lanes
    NW = NC * NS
    EPW = B // NW

    mesh = plsc.VectorSubcoreMesh(core_axis_name="c", subcore_axis_name="s")

    @functools.partial(
        pl.kernel,
        mesh=mesh,
        compiler_params=pltpu.CompilerParams(use_tc_tiling_on_sc=False),
        out_type=jax.ShapeDtypeStruct((B * S * D // 128, 128), jnp.float32),
        scratch_types=[
            pltpu.VMEM((EPW, S), jnp.int32),
            pltpu.VMEM((2, S, D), jnp.float32),
            pltpu.VMEM((2, S * D // 128, 128), jnp.float32),
            pltpu.VMEM((S, D), jnp.float32),
        ]
        + [pltpu.SemaphoreType.DMA] * 4,
    )
    def emb_kernel(x_hbm, table_hbm, pos_hbm, out_hbm, idx_all, g_v, o_v, pos_v, *sems):
        gsem = sems[:2]
        ssem = sems[2:]
        wid = lax.axis_index("s") * NC + lax.axis_index("c")
        e0 = wid * EPW

        pltpu.sync_copy(pos_hbm, pos_v)
        pltpu.sync_copy(x_hbm.at[pl.ds(e0, EPW)], idx_all)

        def start_gather(c, b):
            pltpu.async_copy(table_hbm.at[idx_all.at[c]], g_v.at[b], gsem[b])

        def wait_gather(c, b):
            pltpu.make_async_copy(table_hbm.at[idx_all.at[c]], g_v.at[b], gsem[b]).wait()

        RPE = S * D // 128

        def start_store(c, b):
            pltpu.async_copy(o_v.at[b], out_hbm.at[pl.ds((e0 + c) * RPE, RPE)], ssem[b])

        def wait_store(c, b):
            pltpu.make_async_copy(
                o_v.at[b], out_hbm.at[pl.ds((e0 + c) * RPE, RPE)], ssem[b]
            ).wait()

        start_gather(0, 0)

        @pl.loop(0, EPW, step=2)
        def ring(g):
            for k in range(2):
                c = g + k
                b = k

                @pl.when(c + 1 < EPW)
                def _():
                    start_gather(c + 1, 1 - b)

                wait_gather(c, b)

                @pl.when(c >= 2)
                def _():
                    wait_store(c - 2, b)

                @pl.loop(0, S // 2)
                def row_add(q):
                    for p in range(2):
                        for d in range(D // L):
                            sl = pl.ds(d * L, L)
                            o_v[b, q, pl.ds(p * D + d * L, L)] = (
                                g_v[b, 2 * q + p, sl] + pos_v[2 * q + p, sl]
                            )

                start_store(c, b)

        wait_store(EPW - 2, EPW % 2)
        wait_store(EPW - 1, 1 - EPW % 2)

    tbl_pairs = jax.lax.optimization_barrier(emb_table.reshape(V // 2, 2 * D))
    tbl_lin = tbl_pairs.reshape(V, D)
    out = emb_kernel(x.astype(jnp.int32), tbl_lin, pos_embd)
    out2 = jax.lax.optimization_barrier(out.reshape(B, S * D))
    return out2.reshape(B, S, D)
